# baseline (speedup 1.0000x reference)
import functools

import jax
import jax.numpy as jnp
from jax import lax
from jax.experimental import pallas as pl
from jax.experimental.pallas import tpu as pltpu
from jax.experimental.pallas import tpu_sc as plsc

BATCH = 16384
DIM = 64
NUM_CORES = 2
NUM_SUBCORES = 16
LANES = 16
NW = NUM_CORES * NUM_SUBCORES
BPW = BATCH // NW
CHUNK = 128
NCHUNK = BPW // CHUNK


def _mf_body(users_hbm, items_hbm, eu_hbm, ei_hbm, out_hbm,
             idx_u, idx_i, rows_u, rows_i, out_v, sems):
    wid = lax.axis_index("s") * NUM_CORES + lax.axis_index("c")
    base = wid * BPW

    pltpu.sync_copy(users_hbm.at[pl.ds(base, BPW)], idx_u)
    pltpu.sync_copy(items_hbm.at[pl.ds(base, BPW)], idx_i)

    def fire(k):
        for j in range(DIM):
            pltpu.async_copy(eu_hbm.at[j].at[idx_u.at[pl.ds(k * CHUNK, CHUNK)]],
                             rows_u.at[j].at[pl.ds(k * CHUNK, CHUNK)],
                             sems.at[k % 2])
            pltpu.async_copy(ei_hbm.at[j].at[idx_i.at[pl.ds(k * CHUNK, CHUNK)]],
                             rows_i.at[j].at[pl.ds(k * CHUNK, CHUNK)],
                             sems.at[k % 2])

    def drain(k):
        pltpu.make_async_copy(eu_hbm.at[:, pl.ds(0, CHUNK)],
                              rows_u.at[:, pl.ds(k * CHUNK, CHUNK)],
                              sems.at[k % 2]).wait()
        pltpu.make_async_copy(ei_hbm.at[:, pl.ds(0, CHUNK)],
                              rows_i.at[:, pl.ds(k * CHUNK, CHUNK)],
                              sems.at[k % 2]).wait()

    def compute(k):
        def blk(c, carry):
            col = k * CHUNK + c * LANES
            acc = rows_u[0, pl.ds(col, LANES)] * rows_i[0, pl.ds(col, LANES)]
            for j in range(1, DIM):
                acc = acc + (rows_u[j, pl.ds(col, LANES)] *
                             rows_i[j, pl.ds(col, LANES)])
            out_v[pl.ds(col, LANES)] = 1.0 / (1.0 + jnp.exp(-acc))
            return carry
        lax.fori_loop(0, CHUNK // LANES, blk, 0)

    fire(0)
    for k in range(1, NCHUNK):
        fire(k)
        drain(k - 1)
        compute(k - 1)
    drain(NCHUNK - 1)
    compute(NCHUNK - 1)

    pltpu.sync_copy(out_v, out_hbm.at[pl.ds(base, BPW)])


_mf = functools.partial(
    pl.kernel,
    mesh=plsc.VectorSubcoreMesh(core_axis_name="c", subcore_axis_name="s"),
    compiler_params=pltpu.CompilerParams(needs_layout_passes=False,
                                         use_tc_tiling_on_sc=False),
    out_type=jax.ShapeDtypeStruct((BATCH,), jnp.float32),
    scratch_types=[
        pltpu.VMEM((BPW,), jnp.int32),
        pltpu.VMEM((BPW,), jnp.int32),
        pltpu.VMEM((DIM, BPW), jnp.float32),
        pltpu.VMEM((DIM, BPW), jnp.float32),
        pltpu.VMEM((BPW,), jnp.float32),
        pltpu.SemaphoreType.DMA((2,)),
    ],
)(_mf_body)


def kernel(users, items, embedding_user, embedding_item):
    eu_t = embedding_user.T
    ei_t = embedding_item.T
    return _mf(users.astype(jnp.int32), items.astype(jnp.int32), eu_t, ei_t)


# speedup vs baseline: 19.8344x; 19.8344x over previous
import functools

import jax
import jax.numpy as jnp
from jax import lax
from jax.experimental import pallas as pl
from jax.experimental.pallas import tpu as pltpu
from jax.experimental.pallas import tpu_sc as plsc

BATCH = 16384
DIM = 64
NUM_CORES = 2
NUM_SUBCORES = 16
LANES = 16
NW = NUM_CORES * NUM_SUBCORES
BPW = BATCH // NW
BLK = 128
NBLK = BPW // BLK
NBAND = 8
GRP = 16
NGRP = BLK // GRP
NSLOT = 64
PDIM = 2 * DIM


def _extract_body(su_hbm, si_hbm, eu_hbm, ei_hbm, gu_hbm, gi_hbm,
                  idxv, ring, rbuf, sem):
    wid = lax.axis_index("s") * NUM_CORES + lax.axis_index("c")
    base = wid * BPW
    lane_iota = lax.iota(jnp.int32, LANES)
    perm = lax.rem(lane_iota + 15, jnp.full((LANES,), 16, jnp.int32))
    lane0 = lane_iota == 0
    feat_pat = lax.rem(lane_iota, jnp.full((LANES,), NBAND, jnp.int32))
    low8 = lane_iota < NBAND

    for idx_hbm, tbl_hbm, out_hbm in ((su_hbm, eu_hbm, gu_hbm),
                                      (si_hbm, ei_hbm, gi_hbm)):
        pltpu.sync_copy(idx_hbm.at[pl.ds(base, BPW)], idxv)

        def body(it, carry, tbl_hbm=tbl_hbm, out_hbm=out_hbm):
            S0, = carry
            blk = it // NBAND
            band = it % NBAND
            brow = pl.multiple_of(band * NBAND, NBAND)

            def group_info(g, S):
                iv = idxv[pl.ds(blk * BLK + g * GRP, GRP)]
                tv = iv // 128
                lanes = lax.rem(iv, jnp.full((GRP,), 128, jnp.int32))
                shifted = tv.at[perm].get(mode="promise_in_bounds")
                if g == 0:
                    inew = jnp.where(lane0, 1,
                                     (tv != shifted).astype(jnp.int32))
                else:
                    prev = idxv[pl.ds(blk * BLK + g * GRP - 16, GRP)] // 128
                    pshift = prev.at[perm].get(mode="promise_in_bounds")
                    shifted = jnp.where(lane0, pshift, shifted)
                    inew = (tv != shifted).astype(jnp.int32)
                slots = S + jnp.cumsum(inew) - 1
                return tv, lanes, inew, slots, slots[15] + 1

            def fire(tv, inew, slots):
                for t in range(GRP):
                    @pl.when(inew[t] == 1)
                    def _(t=t):
                        tcol = pl.multiple_of(tv[t] * 128, 128)
                        pltpu.async_copy(
                            tbl_hbm.at[pl.ds(brow, NBAND), pl.ds(tcol, 128)],
                            ring.at[lax.rem(slots[t], jnp.int32(NSLOT))], sem)

            def drain(n):
                def w(i, c):
                    pltpu.make_async_copy(
                        tbl_hbm.at[pl.ds(0, NBAND), pl.ds(0, 128)],
                        ring.at[0], sem).wait()
                    return c
                lax.fori_loop(0, n, w, 0)

            def extract(g, lanes, slots):
                for t in range(GRP):
                    sl = lax.rem(slots[t], jnp.int32(NSLOT))
                    vals = plsc.load_gather(
                        ring, [jnp.zeros((GRP,), jnp.int32) + sl, feat_pat,
                               jnp.zeros((GRP,), jnp.int32) + lanes[t]])
                    vals = jnp.where(low8, vals, 0.0)
                    rbuf[g * GRP + t, pl.ds(band * GRP, GRP)] = vals

            infos = []
            S = S0
            for g in range(2):
                tv, lanes, inew, slots, S = group_info(g, S)
                fire(tv, inew, slots)
                infos.append((lanes, slots, S))
            D = S0
            for g in range(NGRP):
                if g + 2 < NGRP:
                    tv, lanes, inew, slots, S = group_info(g + 2, S)
                    fire(tv, inew, slots)
                    infos.append((lanes, slots, S))
                lanes_g, slots_g, S_g = infos[g]
                drain(S_g - D)
                D = S_g
                extract(g, lanes_g, slots_g)

            @pl.when(band == NBAND - 1)
            def _():
                pltpu.sync_copy(
                    rbuf, out_hbm.at[pl.ds(base + blk * BLK, BLK)])
            return (S,)

        lax.fori_loop(0, NBLK * NBAND, body, (jnp.int32(0),))


def _dot_body(invu_hbm, invi_hbm, gu_hbm, gi_hbm, out_hbm,
              idx_u, idx_i, rows_u, rows_i, out_v, sem_u, sem_i):
    wid = lax.axis_index("s") * NUM_CORES + lax.axis_index("c")
    lane_ids = lax.iota(jnp.int32, LANES)
    pltpu.sync_copy(invu_hbm.at[pl.ds(wid * NBLK, NBLK)], idx_u)
    pltpu.sync_copy(invi_hbm.at[pl.ds(wid * NBLK, NBLK)], idx_i)

    def fire(k):
        s = k % 2
        cu = pltpu.async_copy(gu_hbm.at[idx_u.at[k]], rows_u.at[s], sem_u)
        ci = pltpu.async_copy(gi_hbm.at[idx_i.at[k]], rows_i.at[s], sem_i)
        return cu, ci

    copies = [fire(0), fire(1)]
    for k in range(NBLK):
        s = k % 2
        cu, ci = copies[k]
        cu.wait()
        ci.wait()

        def block_body(b, carry, s=s, k=k):
            out_acc = jnp.zeros((LANES,), jnp.float32)
            for rr in range(LANES):
                r = b * LANES + rr
                acc = (rows_u[s, r, pl.ds(0, LANES)] *
                       rows_i[s, r, pl.ds(0, LANES)])
                for q in range(1, PDIM // LANES):
                    acc = acc + (rows_u[s, r, pl.ds(q * LANES, LANES)] *
                                 rows_i[s, r, pl.ds(q * LANES, LANES)])
                out_acc = jnp.where(lane_ids == rr, jnp.sum(acc), out_acc)
            out_v[pl.ds(k * BLK + b * LANES, LANES)] = (
                1.0 / (1.0 + jnp.exp(-out_acc)))
            return carry

        lax.fori_loop(0, BLK // LANES, block_body, 0)
        if k + 2 < NBLK:
            copies.append(fire(k + 2))

    pltpu.sync_copy(out_v, out_hbm.at[pl.ds(wid * BPW, BPW)])


_extract = functools.partial(
    pl.kernel,
    mesh=plsc.VectorSubcoreMesh(core_axis_name="c", subcore_axis_name="s"),
    compiler_params=pltpu.CompilerParams(needs_layout_passes=False,
                                         use_tc_tiling_on_sc=True),
    out_type=(jax.ShapeDtypeStruct((BATCH, PDIM), jnp.float32),
              jax.ShapeDtypeStruct((BATCH, PDIM), jnp.float32)),
    scratch_types=[
        pltpu.VMEM((BPW,), jnp.int32),
        pltpu.VMEM((NSLOT, NBAND, 128), jnp.float32),
        pltpu.VMEM((BLK, PDIM), jnp.float32),
        pltpu.SemaphoreType.DMA,
    ],
)(_extract_body)

_dot = functools.partial(
    pl.kernel,
    mesh=plsc.VectorSubcoreMesh(core_axis_name="c", subcore_axis_name="s"),
    compiler_params=pltpu.CompilerParams(needs_layout_passes=False,
                                         use_tc_tiling_on_sc=False),
    out_type=jax.ShapeDtypeStruct((BATCH,), jnp.float32),
    scratch_types=[
        pltpu.VMEM((NBLK, BLK), jnp.int32),
        pltpu.VMEM((NBLK, BLK), jnp.int32),
        pltpu.VMEM((2, BLK, PDIM), jnp.float32),
        pltpu.VMEM((2, BLK, PDIM), jnp.float32),
        pltpu.VMEM((BPW,), jnp.float32),
        pltpu.SemaphoreType.DMA,
        pltpu.SemaphoreType.DMA,
    ],
)(_dot_body)


def kernel(users, items, embedding_user, embedding_item):
    u = users.astype(jnp.int32)
    it = items.astype(jnp.int32)
    eu_t = embedding_user.T
    ei_t = embedding_item.T
    ou = jnp.argsort(u).astype(jnp.int32)
    oi = jnp.argsort(it).astype(jnp.int32)
    su = jnp.take(u, ou)
    si = jnp.take(it, oi)
    ar = jnp.arange(BATCH, dtype=jnp.int32)
    inv_u = jnp.zeros((BATCH,), jnp.int32).at[ou].set(ar)
    inv_i = jnp.zeros((BATCH,), jnp.int32).at[oi].set(ar)
    gu, gi = _extract(su, si, eu_t, ei_t)
    return _dot(inv_u.reshape(NW * NBLK, BLK), inv_i.reshape(NW * NBLK, BLK),
                gu, gi)


# speedup vs baseline: 32.4283x; 1.6350x over previous
import functools

import jax
import jax.numpy as jnp
from jax import lax
from jax.experimental import pallas as pl
from jax.experimental.pallas import tpu as pltpu
from jax.experimental.pallas import tpu_sc as plsc

BATCH = 16384
DIM = 64
NUM_CORES = 2
NUM_SUBCORES = 16
LANES = 16
NW = NUM_CORES * NUM_SUBCORES
BPW = BATCH // NW
BLK = 128
NBLK = BPW // BLK
NBAND = 8
GRP = 16
NGRP = BLK // GRP
NSLOT = 64
PDIM = 2 * DIM


def _extract_body(su_hbm, si_hbm, eu_hbm, ei_hbm, gu_hbm, gi_hbm,
                  idxv, ring, rbuf, sem):
    wid = lax.axis_index("s") * NUM_CORES + lax.axis_index("c")
    base = wid * BPW
    lane_iota = lax.iota(jnp.int32, LANES)
    perm = lax.rem(lane_iota + 15, jnp.full((LANES,), 16, jnp.int32))
    lane0_i = (lane_iota == 0).astype(jnp.int32)
    NIT = NBLK * NBAND * NGRP

    def z(i, c):
        rbuf[i // 8, pl.ds(lax.rem(i, jnp.int32(8)) * LANES, LANES)] = (
            jnp.zeros((LANES,), jnp.float32))
        return c
    lax.fori_loop(0, BLK * (PDIM // LANES), z, 0)

    for idx_hbm, tbl_hbm, out_hbm in ((su_hbm, eu_hbm, gu_hbm),
                                      (si_hbm, ei_hbm, gi_hbm)):
        pltpu.sync_copy(idx_hbm.at[pl.ds(base, BPW)], idxv)

        def decode(it):
            blk = it // (NBAND * NGRP)
            r = lax.rem(it, jnp.int32(NBAND * NGRP))
            band = r // NGRP
            g = lax.rem(r, jnp.int32(NGRP))
            return blk, band, g

        def ginfo_fire(it, S, active, tbl_hbm=tbl_hbm):
            blk, band, g = decode(it)
            off = blk * BLK + g * GRP
            iv = idxv[pl.ds(off, GRP)]
            tv = iv // 128
            lanes = lax.rem(iv, jnp.full((GRP,), 128, jnp.int32))
            shifted = tv.at[perm].get(mode="promise_in_bounds")
            poff = jnp.maximum(off - GRP, 0)
            prev = idxv[pl.ds(poff, GRP)] // 128
            pshift = prev.at[perm].get(mode="promise_in_bounds")
            shifted = jnp.where(lane_iota == 0, pshift, shifted)
            inew = (tv != shifted).astype(jnp.int32)
            force0 = (g == 0).astype(jnp.int32)
            inew = jnp.maximum(inew, lane0_i * force0)
            inew = inew * active
            slots = S + jnp.cumsum(inew) - 1
            brow = pl.multiple_of(band * NBAND, NBAND)
            for t in range(GRP):
                @pl.when(inew[t] == 1)
                def _(t=t):
                    tcol = pl.multiple_of(tv[t] * 128, 128)
                    pltpu.async_copy(
                        tbl_hbm.at[pl.ds(brow, NBAND), pl.ds(tcol, 128)],
                        ring.at[lax.rem(slots[t], jnp.int32(NSLOT))], sem)
            return lanes, slots, slots[GRP - 1] + 1

        def drain(n, tbl_hbm=tbl_hbm):
            def w(i, c):
                pltpu.make_async_copy(
                    tbl_hbm.at[pl.ds(0, NBAND), pl.ds(0, 128)],
                    ring.at[0], sem).wait()
                return c
            lax.fori_loop(0, n, w, 0)

        def extract(it, lanes, slots):
            blk, band, g = decode(it)
            smod = lax.rem(slots, jnp.full((GRP,), NSLOT, jnp.int32))
            rows = g * GRP + lane_iota
            for f in range(NBAND):
                vals = plsc.load_gather(
                    ring, [smod, jnp.full((GRP,), f, jnp.int32), lanes])
                plsc.store_scatter(
                    rbuf, [rows, band * GRP + jnp.full((GRP,), f, jnp.int32)],
                    vals)
            return blk, band, g

        lA, sA, aA = ginfo_fire(jnp.int32(0), jnp.int32(0), jnp.int32(1))
        lB, sB, aB = ginfo_fire(jnp.int32(1), aA, jnp.int32(1))

        def body(it, carry, out_hbm=out_hbm):
            D, lA, sA, aA, lB, sB, aB = carry
            itF = jnp.minimum(it + 2, jnp.int32(NIT - 1))
            active = (it + 2 < NIT).astype(jnp.int32)
            lF, sF, aF = ginfo_fire(itF, aB, active)
            drain(aA - D)
            blk, band, g = extract(it, lA, sA)

            @pl.when(jnp.logical_and(band == NBAND - 1, g == NGRP - 1))
            def _():
                pltpu.sync_copy(rbuf,
                                out_hbm.at[pl.ds(base + blk * BLK, BLK)])
            return (aA, lB, sB, aB, lF, sF, aF)

        lax.fori_loop(0, NIT, body,
                      (jnp.int32(0), lA, sA, aA, lB, sB, aB))


def _dot_body(invu_hbm, invi_hbm, gu_hbm, gi_hbm, out_hbm,
              idx_u, idx_i, rows_u, rows_i, out_v, sem_u, sem_i):
    wid = lax.axis_index("s") * NUM_CORES + lax.axis_index("c")
    lane_ids = lax.iota(jnp.int32, LANES)
    pltpu.sync_copy(invu_hbm.at[pl.ds(wid * NBLK, NBLK)], idx_u)
    pltpu.sync_copy(invi_hbm.at[pl.ds(wid * NBLK, NBLK)], idx_i)

    def fire(k):
        s = k % 2
        cu = pltpu.async_copy(gu_hbm.at[idx_u.at[k]], rows_u.at[s], sem_u)
        ci = pltpu.async_copy(gi_hbm.at[idx_i.at[k]], rows_i.at[s], sem_i)
        return cu, ci

    copies = [fire(0), fire(1)]
    for k in range(NBLK):
        s = k % 2
        cu, ci = copies[k]
        cu.wait()
        ci.wait()

        def block_body(b, carry, s=s, k=k):
            out_acc = jnp.zeros((LANES,), jnp.float32)
            for rr in range(LANES):
                r = b * LANES + rr
                acc = (rows_u[s, r, pl.ds(0, LANES)] *
                       rows_i[s, r, pl.ds(0, LANES)])
                for q in range(1, PDIM // LANES):
                    acc = acc + (rows_u[s, r, pl.ds(q * LANES, LANES)] *
                                 rows_i[s, r, pl.ds(q * LANES, LANES)])
                out_acc = jnp.where(lane_ids == rr, jnp.sum(acc), out_acc)
            out_v[pl.ds(k * BLK + b * LANES, LANES)] = (
                1.0 / (1.0 + jnp.exp(-out_acc)))
            return carry

        lax.fori_loop(0, BLK // LANES, block_body, 0)
        if k + 2 < NBLK:
            copies.append(fire(k + 2))

    pltpu.sync_copy(out_v, out_hbm.at[pl.ds(wid * BPW, BPW)])


_extract = functools.partial(
    pl.kernel,
    mesh=plsc.VectorSubcoreMesh(core_axis_name="c", subcore_axis_name="s"),
    compiler_params=pltpu.CompilerParams(needs_layout_passes=False,
                                         use_tc_tiling_on_sc=True),
    out_type=(jax.ShapeDtypeStruct((BATCH, PDIM), jnp.float32),
              jax.ShapeDtypeStruct((BATCH, PDIM), jnp.float32)),
    scratch_types=[
        pltpu.VMEM((BPW,), jnp.int32),
        pltpu.VMEM((NSLOT, NBAND, 128), jnp.float32),
        pltpu.VMEM((BLK, PDIM), jnp.float32),
        pltpu.SemaphoreType.DMA,
    ],
)(_extract_body)

_dot = functools.partial(
    pl.kernel,
    mesh=plsc.VectorSubcoreMesh(core_axis_name="c", subcore_axis_name="s"),
    compiler_params=pltpu.CompilerParams(needs_layout_passes=False,
                                         use_tc_tiling_on_sc=False),
    out_type=jax.ShapeDtypeStruct((BATCH,), jnp.float32),
    scratch_types=[
        pltpu.VMEM((NBLK, BLK), jnp.int32),
        pltpu.VMEM((NBLK, BLK), jnp.int32),
        pltpu.VMEM((2, BLK, PDIM), jnp.float32),
        pltpu.VMEM((2, BLK, PDIM), jnp.float32),
        pltpu.VMEM((BPW,), jnp.float32),
        pltpu.SemaphoreType.DMA,
        pltpu.SemaphoreType.DMA,
    ],
)(_dot_body)


def kernel(users, items, embedding_user, embedding_item):
    u = users.astype(jnp.int32)
    it = items.astype(jnp.int32)
    eu_t = embedding_user.T
    ei_t = embedding_item.T
    ou = jnp.argsort(u).astype(jnp.int32)
    oi = jnp.argsort(it).astype(jnp.int32)
    su = jnp.take(u, ou)
    si = jnp.take(it, oi)
    ar = jnp.arange(BATCH, dtype=jnp.int32)
    inv_u = jnp.zeros((BATCH,), jnp.int32).at[ou].set(ar)
    inv_i = jnp.zeros((BATCH,), jnp.int32).at[oi].set(ar)
    gu, gi = _extract(su, si, eu_t, ei_t)
    return _dot(inv_u.reshape(NW * NBLK, BLK), inv_i.reshape(NW * NBLK, BLK),
                gu, gi)


# speedup vs baseline: 34.0678x; 1.0506x over previous
import functools

import jax
import jax.numpy as jnp
from jax import lax
from jax.experimental import pallas as pl
from jax.experimental.pallas import tpu as pltpu
from jax.experimental.pallas import tpu_sc as plsc

BATCH = 16384
DIM = 64
NUM_CORES = 2
NUM_SUBCORES = 16
LANES = 16
NW = NUM_CORES * NUM_SUBCORES
BPW = BATCH // NW
BLK = 128
NBLK = BPW // BLK
NBAND = 8
GRP = 16
NGRP = BLK // GRP
NSLOT = 64


def _extract_body(su_hbm, si_hbm, eu_hbm, ei_hbm, gu_hbm, gi_hbm,
                  idxv, ring, rbuf, sem):
    wid = lax.axis_index("s") * NUM_CORES + lax.axis_index("c")
    base = wid * BPW
    lane_iota = lax.iota(jnp.int32, LANES)
    perm = lax.rem(lane_iota + 15, jnp.full((LANES,), 16, jnp.int32))
    lane0_i = (lane_iota == 0).astype(jnp.int32)
    NIT = NBLK * NBAND * NGRP

    for idx_hbm, tbl_hbm, out_hbm in ((su_hbm, eu_hbm, gu_hbm),
                                      (si_hbm, ei_hbm, gi_hbm)):
        pltpu.sync_copy(idx_hbm.at[pl.ds(base, BPW)], idxv)

        def decode(it):
            blk = it // (NBAND * NGRP)
            r = lax.rem(it, jnp.int32(NBAND * NGRP))
            band = r // NGRP
            g = lax.rem(r, jnp.int32(NGRP))
            return blk, band, g

        def ginfo_fire(it, S, active, tbl_hbm=tbl_hbm):
            blk, band, g = decode(it)
            off = blk * BLK + g * GRP
            iv = idxv[pl.ds(off, GRP)]
            tv = iv // 128
            lanes = lax.rem(iv, jnp.full((GRP,), 128, jnp.int32))
            shifted = tv.at[perm].get(mode="promise_in_bounds")
            poff = jnp.maximum(off - GRP, 0)
            prev = idxv[pl.ds(poff, GRP)] // 128
            pshift = prev.at[perm].get(mode="promise_in_bounds")
            shifted = jnp.where(lane_iota == 0, pshift, shifted)
            inew = (tv != shifted).astype(jnp.int32)
            force0 = (g == 0).astype(jnp.int32)
            inew = jnp.maximum(inew, lane0_i * force0)
            inew = inew * active
            slots = S + jnp.cumsum(inew) - 1
            brow = pl.multiple_of(band * NBAND, NBAND)
            for t in range(GRP):
                @pl.when(inew[t] == 1)
                def _(t=t):
                    tcol = pl.multiple_of(tv[t] * 128, 128)
                    pltpu.async_copy(
                        tbl_hbm.at[pl.ds(brow, NBAND), pl.ds(tcol, 128)],
                        ring.at[lax.rem(slots[t], jnp.int32(NSLOT))], sem)
            return lanes, slots, slots[GRP - 1] + 1

        def drain(n, tbl_hbm=tbl_hbm):
            def w(i, c):
                pltpu.make_async_copy(
                    tbl_hbm.at[pl.ds(0, NBAND), pl.ds(0, 128)],
                    ring.at[0], sem).wait()
                return c
            lax.fori_loop(0, n, w, 0)

        def extract(it, lanes, slots):
            blk, band, g = decode(it)
            smod = lax.rem(slots, jnp.full((GRP,), NSLOT, jnp.int32))
            rows = g * GRP + lane_iota
            for f in range(NBAND):
                vals = plsc.load_gather(
                    ring, [smod, jnp.full((GRP,), f, jnp.int32), lanes])
                plsc.store_scatter(
                    rbuf, [rows, band * NBAND + jnp.full((GRP,), f, jnp.int32)],
                    vals)
            return blk, band, g

        lA, sA, aA = ginfo_fire(jnp.int32(0), jnp.int32(0), jnp.int32(1))
        lB, sB, aB = ginfo_fire(jnp.int32(1), aA, jnp.int32(1))

        def body(it, carry, out_hbm=out_hbm):
            D, lA, sA, aA, lB, sB, aB = carry
            itF = jnp.minimum(it + 2, jnp.int32(NIT - 1))
            active = (it + 2 < NIT).astype(jnp.int32)
            lF, sF, aF = ginfo_fire(itF, aB, active)
            drain(aA - D)
            blk, band, g = extract(it, lA, sA)

            @pl.when(jnp.logical_and(band == NBAND - 1, g == NGRP - 1))
            def _():
                pltpu.sync_copy(rbuf,
                                out_hbm.at[pl.ds(base + blk * BLK, BLK)])
            return (aA, lB, sB, aB, lF, sF, aF)

        lax.fori_loop(0, NIT, body,
                      (jnp.int32(0), lA, sA, aA, lB, sB, aB))


def _dot_body(invu_hbm, invi_hbm, gu_hbm, gi_hbm, out_hbm,
              idx_u, idx_i, rows_u, rows_i, out_v, sem_u, sem_i):
    wid = lax.axis_index("s") * NUM_CORES + lax.axis_index("c")
    lane_ids = lax.iota(jnp.int32, LANES)
    pltpu.sync_copy(invu_hbm.at[pl.ds(wid * NBLK, NBLK)], idx_u)
    pltpu.sync_copy(invi_hbm.at[pl.ds(wid * NBLK, NBLK)], idx_i)

    def fire(k):
        s = k % 2
        cu = pltpu.async_copy(gu_hbm.at[idx_u.at[k]], rows_u.at[s], sem_u)
        ci = pltpu.async_copy(gi_hbm.at[idx_i.at[k]], rows_i.at[s], sem_i)
        return cu, ci

    copies = [fire(0), fire(1)]
    for k in range(NBLK):
        s = k % 2
        cu, ci = copies[k]
        cu.wait()
        ci.wait()

        def block_body(b, carry, s=s, k=k):
            out_acc = jnp.zeros((LANES,), jnp.float32)
            for rr in range(LANES):
                r = b * LANES + rr
                acc = (rows_u[s, r, pl.ds(0, LANES)] *
                       rows_i[s, r, pl.ds(0, LANES)])
                for q in range(1, DIM // LANES):
                    acc = acc + (rows_u[s, r, pl.ds(q * LANES, LANES)] *
                                 rows_i[s, r, pl.ds(q * LANES, LANES)])
                out_acc = jnp.where(lane_ids == rr, jnp.sum(acc), out_acc)
            out_v[pl.ds(k * BLK + b * LANES, LANES)] = (
                1.0 / (1.0 + jnp.exp(-out_acc)))
            return carry

        lax.fori_loop(0, BLK // LANES, block_body, 0)
        if k + 2 < NBLK:
            copies.append(fire(k + 2))

    pltpu.sync_copy(out_v, out_hbm.at[pl.ds(wid * BPW, BPW)])


_extract = functools.partial(
    pl.kernel,
    mesh=plsc.VectorSubcoreMesh(core_axis_name="c", subcore_axis_name="s"),
    compiler_params=pltpu.CompilerParams(needs_layout_passes=False,
                                         use_tc_tiling_on_sc=True),
    out_type=(jax.ShapeDtypeStruct((BATCH, DIM), jnp.float32),
              jax.ShapeDtypeStruct((BATCH, DIM), jnp.float32)),
    scratch_types=[
        pltpu.VMEM((BPW,), jnp.int32),
        pltpu.VMEM((NSLOT, NBAND, 128), jnp.float32),
        pltpu.VMEM((BLK, DIM), jnp.float32),
        pltpu.SemaphoreType.DMA,
    ],
)(_extract_body)

_dot = functools.partial(
    pl.kernel,
    mesh=plsc.VectorSubcoreMesh(core_axis_name="c", subcore_axis_name="s"),
    compiler_params=pltpu.CompilerParams(needs_layout_passes=False,
                                         use_tc_tiling_on_sc=False),
    out_type=jax.ShapeDtypeStruct((BATCH,), jnp.float32),
    scratch_types=[
        pltpu.VMEM((NBLK, BLK), jnp.int32),
        pltpu.VMEM((NBLK, BLK), jnp.int32),
        pltpu.VMEM((2, BLK, DIM), jnp.float32),
        pltpu.VMEM((2, BLK, DIM), jnp.float32),
        pltpu.VMEM((BPW,), jnp.float32),
        pltpu.SemaphoreType.DMA,
        pltpu.SemaphoreType.DMA,
    ],
)(_dot_body)


def kernel(users, items, embedding_user, embedding_item):
    u = users.astype(jnp.int32)
    it = items.astype(jnp.int32)
    eu_t = embedding_user.T
    ei_t = embedding_item.T
    ou = jnp.argsort(u).astype(jnp.int32)
    oi = jnp.argsort(it).astype(jnp.int32)
    su = jnp.take(u, ou)
    si = jnp.take(it, oi)
    ar = jnp.arange(BATCH, dtype=jnp.int32)
    inv_u = jnp.zeros((BATCH,), jnp.int32).at[ou].set(ar)
    inv_i = jnp.zeros((BATCH,), jnp.int32).at[oi].set(ar)
    gu, gi = _extract(su, si, eu_t, ei_t)
    return _dot(inv_u.reshape(NW * NBLK, BLK), inv_i.reshape(NW * NBLK, BLK),
                gu, gi)


# speedup vs baseline: 36.6942x; 1.0771x over previous
import functools

import jax
import jax.numpy as jnp
from jax import lax
from jax.experimental import pallas as pl
from jax.experimental.pallas import tpu as pltpu
from jax.experimental.pallas import tpu_sc as plsc

BATCH = 16384
DIM = 64
NUM_CORES = 2
NUM_SUBCORES = 16
LANES = 16
NW = NUM_CORES * NUM_SUBCORES
BPW = BATCH // NW
BLK = 128
NBLK = BPW // BLK
NBAND = 8
GRP = 16
NGRP = BLK // GRP
NSLOT = 96
QD = 4


def _extract_body(su_hbm, si_hbm, eu_hbm, ei_hbm, gu_hbm, gi_hbm,
                  idxv, ring, rbuf, sem):
    wid = lax.axis_index("s") * NUM_CORES + lax.axis_index("c")
    base = wid * BPW
    lane_iota = lax.iota(jnp.int32, LANES)
    perm = lax.rem(lane_iota + 15, jnp.full((LANES,), 16, jnp.int32))
    lane0_i = (lane_iota == 0).astype(jnp.int32)
    NIT = NBLK * NBAND * NGRP

    for idx_hbm, tbl_hbm, out_hbm in ((su_hbm, eu_hbm, gu_hbm),
                                      (si_hbm, ei_hbm, gi_hbm)):
        pltpu.sync_copy(idx_hbm.at[pl.ds(base, BPW)], idxv)

        def decode(it):
            blk = it // (NBAND * NGRP)
            r = lax.rem(it, jnp.int32(NBAND * NGRP))
            band = r // NGRP
            g = lax.rem(r, jnp.int32(NGRP))
            return blk, band, g

        def ginfo_fire(it, S, active, tbl_hbm=tbl_hbm):
            blk, band, g = decode(it)
            off = blk * BLK + g * GRP
            iv = idxv[pl.ds(off, GRP)]
            tv = iv // 128
            lanes = lax.rem(iv, jnp.full((GRP,), 128, jnp.int32))
            shifted = tv.at[perm].get(mode="promise_in_bounds")
            poff = jnp.maximum(off - GRP, 0)
            prev = idxv[pl.ds(poff, GRP)] // 128
            pshift = prev.at[perm].get(mode="promise_in_bounds")
            shifted = jnp.where(lane_iota == 0, pshift, shifted)
            inew = (tv != shifted).astype(jnp.int32)
            force0 = (g == 0).astype(jnp.int32)
            inew = jnp.maximum(inew, lane0_i * force0)
            inew = inew * active
            slots = S + jnp.cumsum(inew) - 1
            brow = pl.multiple_of(band * NBAND, NBAND)
            for t in range(GRP):
                @pl.when(inew[t] == 1)
                def _(t=t):
                    tcol = pl.multiple_of(tv[t] * 128, 128)
                    pltpu.async_copy(
                        tbl_hbm.at[pl.ds(brow, NBAND), pl.ds(tcol, 128)],
                        ring.at[lax.rem(slots[t], jnp.int32(NSLOT))], sem)
            return lanes, slots, slots[GRP - 1] + 1

        def drain(n, tbl_hbm=tbl_hbm):
            def w(i, c):
                pltpu.make_async_copy(
                    tbl_hbm.at[pl.ds(0, NBAND), pl.ds(0, 128)],
                    ring.at[0], sem).wait()
                return c
            lax.fori_loop(0, n, w, 0)

        def extract(it, lanes, slots):
            blk, band, g = decode(it)
            smod = lax.rem(slots, jnp.full((GRP,), NSLOT, jnp.int32))
            rows = g * GRP + lane_iota
            for f in range(NBAND):
                vals = plsc.load_gather(
                    ring, [smod, jnp.full((GRP,), f, jnp.int32), lanes])
                plsc.store_scatter(
                    rbuf, [rows, band * NBAND + jnp.full((GRP,), f, jnp.int32)],
                    vals)
            return blk, band, g

        queue = []
        S = jnp.int32(0)
        for p in range(QD):
            lp, sp, ap = ginfo_fire(jnp.int32(p), S, jnp.int32(1))
            queue.extend((lp, sp, ap))
            S = ap

        def body(it, carry, out_hbm=out_hbm):
            D = carry[0]
            q = list(carry[1:])
            itF = jnp.minimum(it + QD, jnp.int32(NIT - 1))
            active = (it + QD < NIT).astype(jnp.int32)
            lF, sF, aF = ginfo_fire(itF, q[-1], active)
            lA, sA, aA = q[0], q[1], q[2]
            drain(aA - D)
            blk, band, g = extract(it, lA, sA)

            @pl.when(jnp.logical_and(band == NBAND - 1, g == NGRP - 1))
            def _():
                pltpu.sync_copy(rbuf,
                                out_hbm.at[pl.ds(base + blk * BLK, BLK)])
            return tuple([aA] + q[3:] + [lF, sF, aF])

        lax.fori_loop(0, NIT, body, tuple([jnp.int32(0)] + queue))


def _dot_body(invu_hbm, invi_hbm, gu_hbm, gi_hbm, out_hbm,
              idx_u, idx_i, rows_u, rows_i, out_v, sem_u, sem_i):
    wid = lax.axis_index("s") * NUM_CORES + lax.axis_index("c")
    lane_ids = lax.iota(jnp.int32, LANES)
    pltpu.sync_copy(invu_hbm.at[pl.ds(wid * NBLK, NBLK)], idx_u)
    pltpu.sync_copy(invi_hbm.at[pl.ds(wid * NBLK, NBLK)], idx_i)

    def fire(k):
        s = k % 2
        cu = pltpu.async_copy(gu_hbm.at[idx_u.at[k]], rows_u.at[s], sem_u)
        ci = pltpu.async_copy(gi_hbm.at[idx_i.at[k]], rows_i.at[s], sem_i)
        return cu, ci

    copies = [fire(0), fire(1)]
    for k in range(NBLK):
        s = k % 2
        cu, ci = copies[k]
        cu.wait()
        ci.wait()

        def block_body(b, carry, s=s, k=k):
            out_acc = jnp.zeros((LANES,), jnp.float32)
            for rr in range(LANES):
                r = b * LANES + rr
                acc = (rows_u[s, r, pl.ds(0, LANES)] *
                       rows_i[s, r, pl.ds(0, LANES)])
                for q in range(1, DIM // LANES):
                    acc = acc + (rows_u[s, r, pl.ds(q * LANES, LANES)] *
                                 rows_i[s, r, pl.ds(q * LANES, LANES)])
                out_acc = jnp.where(lane_ids == rr, jnp.sum(acc), out_acc)
            out_v[pl.ds(k * BLK + b * LANES, LANES)] = (
                1.0 / (1.0 + jnp.exp(-out_acc)))
            return carry

        lax.fori_loop(0, BLK // LANES, block_body, 0)
        if k + 2 < NBLK:
            copies.append(fire(k + 2))

    pltpu.sync_copy(out_v, out_hbm.at[pl.ds(wid * BPW, BPW)])


_extract = functools.partial(
    pl.kernel,
    mesh=plsc.VectorSubcoreMesh(core_axis_name="c", subcore_axis_name="s"),
    compiler_params=pltpu.CompilerParams(needs_layout_passes=False,
                                         use_tc_tiling_on_sc=True),
    out_type=(jax.ShapeDtypeStruct((BATCH, DIM), jnp.float32),
              jax.ShapeDtypeStruct((BATCH, DIM), jnp.float32)),
    scratch_types=[
        pltpu.VMEM((BPW,), jnp.int32),
        pltpu.VMEM((NSLOT, NBAND, 128), jnp.float32),
        pltpu.VMEM((BLK, DIM), jnp.float32),
        pltpu.SemaphoreType.DMA,
    ],
)(_extract_body)

_dot = functools.partial(
    pl.kernel,
    mesh=plsc.VectorSubcoreMesh(core_axis_name="c", subcore_axis_name="s"),
    compiler_params=pltpu.CompilerParams(needs_layout_passes=False,
                                         use_tc_tiling_on_sc=False),
    out_type=jax.ShapeDtypeStruct((BATCH,), jnp.float32),
    scratch_types=[
        pltpu.VMEM((NBLK, BLK), jnp.int32),
        pltpu.VMEM((NBLK, BLK), jnp.int32),
        pltpu.VMEM((2, BLK, DIM), jnp.float32),
        pltpu.VMEM((2, BLK, DIM), jnp.float32),
        pltpu.VMEM((BPW,), jnp.float32),
        pltpu.SemaphoreType.DMA,
        pltpu.SemaphoreType.DMA,
    ],
)(_dot_body)


def kernel(users, items, embedding_user, embedding_item):
    u = users.astype(jnp.int32)
    it = items.astype(jnp.int32)
    eu_t = embedding_user.T
    ei_t = embedding_item.T
    ou = jnp.argsort(u).astype(jnp.int32)
    oi = jnp.argsort(it).astype(jnp.int32)
    su = jnp.take(u, ou)
    si = jnp.take(it, oi)
    ar = jnp.arange(BATCH, dtype=jnp.int32)
    inv_u = jnp.zeros((BATCH,), jnp.int32).at[ou].set(ar)
    inv_i = jnp.zeros((BATCH,), jnp.int32).at[oi].set(ar)
    gu, gi = _extract(su, si, eu_t, ei_t)
    return _dot(inv_u.reshape(NW * NBLK, BLK), inv_i.reshape(NW * NBLK, BLK),
                gu, gi)
